# baseline (speedup 1.0000x reference)
import jax
import jax.numpy as jnp
from jax.experimental import pallas as pl
from jax.experimental.pallas import tpu as pltpu

B, N, D = 4, 128, 128
HID, REL = 256, 64
SUB = 32


def _fused_body(p_ref, af_ref, w1_ref, b1_ref, w2_ref, b2_ref,
                w3_ref, b3_ref, w4_ref, b4_ref, out_ref):
    p_all = p_ref[...]
    mflat = (af_ref[...] == 1).astype(jnp.float32)
    w2t = w2_ref[...].T.astype(jnp.bfloat16)
    pt = p_all.T
    w1at = w1_ref[:D, :].T
    w1bt = w1_ref[D:, :].T
    s_t = (jnp.dot(w1at, pt, preferred_element_type=jnp.float32)
           + b1_ref[...]).astype(jnp.bfloat16)
    r_t = jnp.dot(w1bt, pt,
                  preferred_element_type=jnp.float32).astype(jnp.bfloat16)

    b2c = b2_ref[...]
    nch = N // SUB
    rel_blocks = []
    for b in range(B):
        r_tb = r_t[:, b * N:(b + 1) * N]
        contribs = []
        for u in range(nch):
            lo = b * N + u * SUB
            s_blk = s_t[:, lo:lo + SUB]
            h_t = jnp.maximum(s_blk[:, :, None] + r_tb[:, None, :],
                              jnp.bfloat16(0.0))
            f_t = jnp.dot(w2t, h_t.reshape(HID, SUB * N),
                          preferred_element_type=jnp.float32)
            f_t = jnp.maximum(f_t + b2c, 0.0)
            mu = mflat[:, u * SUB * N:(u + 1) * SUB * N]
            fm = (f_t * mu).reshape(REL, SUB, N)
            contribs.append(jnp.sum(fm, axis=1))
        acc_t = sum(contribs)
        rel_blocks.append(acc_t.T)
    rel = jnp.concatenate(rel_blocks, axis=0)

    h2 = jnp.maximum(
        jnp.dot(p_all, w3_ref[:D, :], preferred_element_type=jnp.float32)
        + jnp.dot(rel, w3_ref[D:, :], preferred_element_type=jnp.float32)
        + b3_ref[0][None, :],
        0.0)
    delta = jnp.dot(h2, w4_ref[...], preferred_element_type=jnp.float32) \
        + b4_ref[0][None, :]
    out_ref[...] = p_all + delta


def kernel(particles, adjacency_matrix, W1, b1, W2, b2, W3, b3, W4, b4):
    out = pl.pallas_call(
        _fused_body,
        out_shape=jax.ShapeDtypeStruct((B * N, D), jnp.float32),
    )(particles.reshape(B * N, D), adjacency_matrix.reshape(1, N * N),
      W1, b1.reshape(HID, 1),
      W2, b2.reshape(REL, 1),
      W3, b3.reshape(1, HID), W4, b4.reshape(1, D))
    return out.reshape(B, N, D)


# speedup vs baseline: 1.8746x; 1.8746x over previous
import jax
import jax.numpy as jnp
from jax.experimental import pallas as pl
from jax.experimental.pallas import tpu as pltpu

B, N, D = 4, 128, 128
HID, REL = 256, 64
SUB = 32


def _fused_body(p_ref, a_ref, w1_ref, b1_ref, w2_ref, b2_ref,
                w3_ref, b3_ref, w4_ref, b4_ref, out_ref):
    p_all = p_ref[...]
    a_mask = (a_ref[...] == 1).astype(jnp.float32)
    w2 = w2_ref[...].astype(jnp.bfloat16)
    s_all = (jnp.dot(p_all, w1_ref[:D, :], preferred_element_type=jnp.float32)
             + b1_ref[0][None, :]).astype(jnp.bfloat16)
    r_all = jnp.dot(p_all, w1_ref[D:, :],
                    preferred_element_type=jnp.float32).astype(jnp.bfloat16)

    npair = N // SUB // 2
    m_pairs = []
    for pi in range(npair):
        mu = a_mask[(2 * pi) * SUB:(2 * pi + 1) * SUB, :][:, :, None]
        mv = a_mask[(2 * pi + 1) * SUB:(2 * pi + 2) * SUB, :][:, :, None]
        m_pairs.append(jnp.concatenate(
            [jnp.broadcast_to(mu, (SUB, N, REL)),
             jnp.broadcast_to(mv, (SUB, N, REL))], axis=-1))
    b2v = b2_ref[0][None, :]
    b2p = jnp.concatenate([b2v, b2v], axis=-1)[None]

    rel_blocks = []
    for b in range(B):
        r_b = r_all[b * N:(b + 1) * N, :]
        contribs = []
        for pi in range(npair):
            fs = []
            for u in (2 * pi, 2 * pi + 1):
                lo = b * N + u * SUB
                s_blk = s_all[lo:lo + SUB, :]
                h = jnp.maximum(s_blk[:, None, :] + r_b[None, :, :],
                                jnp.bfloat16(0.0))
                fs.append(jnp.dot(h.reshape(SUB * N, HID), w2,
                                  preferred_element_type=jnp.float32
                                  ).reshape(SUB, N, REL))
            fp = jnp.concatenate(fs, axis=-1) + b2p
            fp = jnp.maximum(fp, 0.0)
            contribs.append(jnp.sum(fp * m_pairs[pi], axis=0))
        acc2 = sum(contribs)
        rel_blocks.append(acc2[:, :REL] + acc2[:, REL:])
    rel = jnp.concatenate(rel_blocks, axis=0)

    h2 = jnp.maximum(
        jnp.dot(p_all, w3_ref[:D, :], preferred_element_type=jnp.float32)
        + jnp.dot(rel, w3_ref[D:, :], preferred_element_type=jnp.float32)
        + b3_ref[0][None, :],
        0.0)
    delta = jnp.dot(h2, w4_ref[...], preferred_element_type=jnp.float32) \
        + b4_ref[0][None, :]
    out_ref[...] = p_all + delta


def kernel(particles, adjacency_matrix, W1, b1, W2, b2, W3, b3, W4, b4):
    out = pl.pallas_call(
        _fused_body,
        out_shape=jax.ShapeDtypeStruct((B * N, D), jnp.float32),
    )(particles.reshape(B * N, D), adjacency_matrix, W1, b1.reshape(1, HID),
      W2, b2.reshape(1, REL),
      W3, b3.reshape(1, HID), W4, b4.reshape(1, D))
    return out.reshape(B, N, D)


# speedup vs baseline: 1.9054x; 1.0164x over previous
import jax
import jax.numpy as jnp
from jax.experimental import pallas as pl
from jax.experimental.pallas import tpu as pltpu

B, N, D = 4, 128, 128
HID, REL = 256, 64
SUB = 32


def _fused_body(p_ref, a_ref, w1_ref, b1_ref, w2_ref, b2_ref,
                w3_ref, b3_ref, w4_ref, b4_ref, out_ref):
    p_all = p_ref[...]
    a_mask = (a_ref[...] == 1).astype(jnp.float32)
    w2 = w2_ref[...].astype(jnp.bfloat16)
    s_all = (jnp.dot(p_all, w1_ref[:D, :], preferred_element_type=jnp.float32)
             + b1_ref[0][None, :]).astype(jnp.bfloat16)
    r_all = jnp.dot(p_all, w1_ref[D:, :],
                    preferred_element_type=jnp.float32).astype(jnp.bfloat16)

    rel_blocks = []
    for b in range(B):
        r_b = r_all[b * N:(b + 1) * N, :]
        contribs = []
        for u in range(N // SUB):
            lo = b * N + u * SUB
            s_blk = s_all[lo:lo + SUB, :]
            h = jnp.maximum(s_blk[:, None, :] + r_b[None, :, :],
                            jnp.bfloat16(0.0))
            f = jnp.dot(h.reshape(SUB * N, HID), w2,
                        preferred_element_type=jnp.float32) + b2_ref[0][None, :]
            f = jnp.maximum(f, 0.0).reshape(SUB, N, REL)
            a_blk = a_mask[u * SUB:(u + 1) * SUB, :]
            contribs.append(jnp.sum(f * a_blk[:, :, None], axis=0))
        rel_blocks.append(sum(contribs))
    rel = jnp.concatenate(rel_blocks, axis=0)

    h2 = jnp.maximum(
        jnp.dot(p_all, w3_ref[:D, :], preferred_element_type=jnp.float32)
        + jnp.dot(rel, w3_ref[D:, :], preferred_element_type=jnp.float32)
        + b3_ref[0][None, :],
        0.0)
    delta = jnp.dot(h2, w4_ref[...], preferred_element_type=jnp.float32) \
        + b4_ref[0][None, :]
    out_ref[...] = p_all + delta


def kernel(particles, adjacency_matrix, W1, b1, W2, b2, W3, b3, W4, b4):
    out = pl.pallas_call(
        _fused_body,
        out_shape=jax.ShapeDtypeStruct((B * N, D), jnp.float32),
    )(particles.reshape(B * N, D), adjacency_matrix, W1, b1.reshape(1, HID),
      W2, b2.reshape(1, REL),
      W3, b3.reshape(1, HID), W4, b4.reshape(1, D))
    return out.reshape(B, N, D)
